# Initial kernel scaffold; baseline (speedup 1.0000x reference)
import jax
import jax.numpy as jnp
from jax.experimental import pallas as pl


def kernel(x, table):
    raise NotImplementedError("write your pallas kernel here")


# speedup vs baseline: 1.0217x; 1.0217x over previous
import functools

import numpy as np
import jax
import jax.numpy as jnp
from jax import lax
from jax.experimental import pallas as pl
from jax.experimental.pallas import tpu as pltpu
from jax.experimental.pallas import tpu_sc as plsc

_MAXLEN = 2048
_D = 128
_B = 4
_BT = _B * _MAXLEN
_NC, _NS, _L = 2, 16, 16
_NW = _NC * _NS
_BPW = _BT // _NW
_CH = 128
_NCH = _BPW // _CH


def _positional_encoding():
    pos = np.arange(_MAXLEN)[:, np.newaxis]
    i = np.arange(_D)[np.newaxis, :]
    angle = pos * (1.0 / np.power(10000, 2 * (i // 2) / np.float32(_D)))
    angle[:, 0::2] = np.sin(angle[:, 0::2])
    angle[:, 1::2] = np.cos(angle[:, 1::2])
    return angle.astype(np.float32)


_POS = _positional_encoding()

_mesh = plsc.VectorSubcoreMesh(core_axis_name="c", subcore_axis_name="s")


@functools.partial(
    pl.kernel,
    mesh=_mesh,
    out_type=jax.ShapeDtypeStruct((_BT, _D), jnp.float32),
    scratch_types=[
        pltpu.VMEM((_NCH, _CH), jnp.int32),
        pltpu.VMEM((_BPW, _D), jnp.float32),
        pltpu.VMEM((_BPW, _D), jnp.float32),
        pltpu.SemaphoreType.DMA,
    ],
)
def _emb_kernel(x_hbm, table_hbm, pos_hbm, out_hbm, idx_v, rows_v, pos_v, gsem):
    wid = lax.axis_index("s") * _NC + lax.axis_index("c")
    base = wid * _BPW
    pltpu.sync_copy(x_hbm.at[pl.ds(wid * _NCH, _NCH)], idx_v)
    handles = []
    for j in range(_NCH):
        handles.append(
            pltpu.async_copy(
                table_hbm.at[idx_v.at[j]],
                rows_v.at[pl.ds(j * _CH, _CH)],
                gsem,
            )
        )
    l0 = lax.rem(base, _MAXLEN)
    pltpu.sync_copy(pos_hbm.at[pl.ds(l0, _BPW)], pos_v)
    for h in handles:
        h.wait()

    def add_row(i, carry):
        for j in range(_D // _L):
            s = pl.ds(j * _L, _L)
            rows_v[i, s] = rows_v[i, s] + pos_v[i, s]
        return carry

    lax.fori_loop(0, _BPW, add_row, 0)
    pltpu.sync_copy(rows_v, out_hbm.at[pl.ds(base, _BPW)])


def kernel(x, table):
    idx = x.reshape(_BT // _CH, _CH).astype(jnp.int32)
    out = _emb_kernel(idx, table, jnp.asarray(_POS))
    return out.reshape(1 * _B, _MAXLEN, _D)


# speedup vs baseline: 1.0977x; 1.0744x over previous
import functools

import numpy as np
import jax
import jax.numpy as jnp
from jax import lax
from jax.experimental import pallas as pl
from jax.experimental.pallas import tpu as pltpu
from jax.experimental.pallas import tpu_sc as plsc

_MAXLEN = 2048
_D = 128
_B = 4
_BT = _B * _MAXLEN
_NC, _NS, _L = 2, 16, 16
_NW = _NC * _NS
_BPW = _BT // _NW
_CH = 128
_NCH = _BPW // _CH


def _positional_encoding():
    pos = np.arange(_MAXLEN)[:, np.newaxis]
    i = np.arange(_D)[np.newaxis, :]
    angle = pos * (1.0 / np.power(10000, 2 * (i // 2) / np.float32(_D)))
    angle[:, 0::2] = np.sin(angle[:, 0::2])
    angle[:, 1::2] = np.cos(angle[:, 1::2])
    return angle.astype(np.float32)


_POS = _positional_encoding()

_mesh = plsc.VectorSubcoreMesh(core_axis_name="c", subcore_axis_name="s")


@functools.partial(
    pl.kernel,
    mesh=_mesh,
    out_type=jax.ShapeDtypeStruct((_BT, _D), jnp.float32),
    scratch_types=[
        pltpu.VMEM((_NCH, _CH), jnp.int32),
        pltpu.VMEM((_BPW, _D), jnp.float32),
        pltpu.SemaphoreType.DMA,
        pltpu.SemaphoreType.DMA,
    ],
)
def _emb_kernel(x_hbm, table_hbm, pos_hbm, out_hbm, idx_v, rows_v, psem, gsem):
    wid = lax.axis_index("s") * _NC + lax.axis_index("c")
    base = wid * _BPW
    l0 = lax.rem(base, _MAXLEN)
    ph = pltpu.async_copy(pos_hbm.at[pl.ds(l0, _BPW)], rows_v, psem)
    pltpu.sync_copy(x_hbm.at[pl.ds(wid * _NCH, _NCH)], idx_v)
    ph.wait()
    handles = []
    for j in range(_NCH):
        handles.append(
            pltpu.async_copy(
                table_hbm.at[idx_v.at[j]],
                rows_v.at[pl.ds(j * _CH, _CH)],
                gsem,
                add=True,
            )
        )
    for h in handles:
        h.wait()
    pltpu.sync_copy(rows_v, out_hbm.at[pl.ds(base, _BPW)])


def kernel(x, table):
    idx = x.reshape(_BT // _CH, _CH).astype(jnp.int32)
    out = _emb_kernel(idx, table, jnp.asarray(_POS))
    return out.reshape(1 * _B, _MAXLEN, _D)
